# baseline (speedup 1.0000x reference)
import jax
import jax.numpy as jnp
from jax import lax
from jax.experimental import pallas as pl
from jax.experimental.pallas import tpu as pltpu
from jax.experimental.pallas import tpu_sc as plsc

B = 16384
N_FIX = 26
N_VAR = 4
VOCAB = 100000
L = 50
D = 32
N_OUT = N_FIX + N_VAR

NC = 2
NS = 16
NW = NC * NS

R = 8
BLKS = B // R
BPW = BLKS // NW
PAIRS = R * N_VAR
VROWS = R * N_VAR * L
VG = 13
VGN = 128
VPAD = VG * VGN
OROWS = R * N_OUT
FG = 2
FGN = OROWS // FG


def _sc_body(wfix_hbm, wvar_hbm, fixidx_hbm, varidx_hbm, out_hbm,
             fixidx_v, varidx_v, var_buf, outbuf,
             isem0, isem1, gsem0, gsem1, osem0, osem1):
  wid = lax.axis_index("s") * NC + lax.axis_index("c")
  blk0 = wid * BPW
  isems = (isem0, isem1)
  gsems = (gsem0, gsem1)
  osems = (osem0, osem1)
  inv_l = jnp.float32(1.0 / L)

  def stage_idx(blk, s):
    pltpu.async_copy(fixidx_hbm.at[pl.ds(blk * OROWS, OROWS)],
                     fixidx_v.at[s], isems[s])
    pltpu.async_copy(varidx_hbm.at[blk], varidx_v.at[s], isems[s])

  def wait_idx(s):
    pltpu.make_async_copy(fixidx_hbm.at[pl.ds(0, OROWS)], fixidx_v.at[s],
                          isems[s]).wait()
    pltpu.make_async_copy(varidx_hbm.at[0], varidx_v.at[s], isems[s]).wait()

  def fire_gathers(s):
    for h in range(FG):
      pltpu.async_copy(
          wfix_hbm.at[fixidx_v.at[s, pl.ds(h * FGN, FGN)]],
          outbuf.at[s, pl.ds(h * FGN, FGN)], gsems[s])
    for g in range(VG):
      pltpu.async_copy(
          wvar_hbm.at[varidx_v.at[s, g]],
          var_buf.at[s, pl.ds(g * VGN, VGN)], gsems[s])

  def drain_gathers(s):
    for h in range(FG):
      pltpu.make_async_copy(
          wfix_hbm.at[fixidx_v.at[s, pl.ds(0, FGN)]],
          outbuf.at[s, pl.ds(0, FGN)], gsems[s]).wait()
    for g in range(VG):
      pltpu.make_async_copy(
          wvar_hbm.at[varidx_v.at[s, 0]],
          var_buf.at[s, pl.ds(0, VGN)], gsems[s]).wait()

  def pool(s):
    for p0 in range(0, PAIRS, 8):
      def red(r, acc):
        new = []
        for k in range(8):
          p = p0 + k
          new.append(acc[2 * k] + var_buf[s, p * L + r, pl.ds(0, 16)])
          new.append(acc[2 * k + 1] + var_buf[s, p * L + r, pl.ds(16, 16)])
        return tuple(new)

      acc = lax.fori_loop(
          0, L, red, tuple(jnp.zeros((16,), jnp.float32) for _ in range(16)))
      for k in range(8):
        p = p0 + k
        row = (p // N_VAR) * N_OUT + N_FIX + (p % N_VAR)
        outbuf[s, row, pl.ds(0, 16)] = acc[2 * k] * inv_l
        outbuf[s, row, pl.ds(16, 16)] = acc[2 * k + 1] * inv_l

  def issue_out(blk, s):
    pltpu.async_copy(outbuf.at[s], out_hbm.at[pl.ds(blk * OROWS, OROWS)],
                     osems[s])

  def wait_out(s):
    pltpu.make_async_copy(outbuf.at[s], out_hbm.at[pl.ds(0, OROWS)],
                          osems[s]).wait()

  stage_idx(blk0, 0)

  def body(i2, carry):
    e = blk0 + 2 * i2
    o = e + 1

    wait_idx(0)

    @pl.when(i2 >= 1)
    def _():
      wait_out(0)

    fire_gathers(0)

    @pl.when(i2 >= 1)
    def _():
      drain_gathers(1)
      stage_idx(o, 1)
      pool(1)
      issue_out(o - 2, 1)

    @pl.when(i2 == 0)
    def _():
      stage_idx(o, 1)

    wait_idx(1)

    @pl.when(i2 >= 1)
    def _():
      wait_out(1)

    fire_gathers(1)
    drain_gathers(0)

    @pl.when(i2 < BPW // 2 - 1)
    def _():
      stage_idx(e + 2, 0)

    pool(0)
    issue_out(e, 0)
    return carry

  lax.fori_loop(0, BPW // 2, body, 0, unroll=False)

  drain_gathers(1)
  pool(1)
  issue_out(blk0 + BPW - 1, 1)
  wait_out(0)
  wait_out(1)


@jax.jit
def kernel(x_fix, x_varlen, W_fix, W_var):
  wfix = W_fix.reshape(N_FIX * VOCAB, D)
  wvar = W_var.reshape(N_VAR * VOCAB, D)
  fix_gidx = (x_fix.astype(jnp.int32)
              + (jnp.arange(N_FIX, dtype=jnp.int32) * VOCAB)[None, :])
  fix_gidx = jnp.pad(fix_gidx.reshape(BLKS, R, N_FIX),
                     ((0, 0), (0, 0), (0, N_OUT - N_FIX)))
  fix_gidx = fix_gidx.reshape(B * N_OUT)
  var_gidx = (x_varlen.astype(jnp.int32)
              + (jnp.arange(N_VAR, dtype=jnp.int32) * VOCAB)[None, :, None])
  var_gidx = var_gidx.reshape(BLKS, VROWS)
  var_gidx = jnp.pad(var_gidx, ((0, 0), (0, VPAD - VROWS)))
  var_gidx = var_gidx.reshape(BLKS, VG, VGN)

  mesh = plsc.VectorSubcoreMesh(core_axis_name="c", subcore_axis_name="s")
  out = pl.kernel(
      _sc_body,
      out_type=jax.ShapeDtypeStruct((B * N_OUT, D), jnp.float32),
      mesh=mesh,
      compiler_params=pltpu.CompilerParams(use_tc_tiling_on_sc=False),
      scratch_types=[
          pltpu.VMEM((2, OROWS), jnp.int32),
          pltpu.VMEM((2, VG, VGN), jnp.int32),
          pltpu.VMEM((2, VPAD, D), jnp.float32),
          pltpu.VMEM((2, OROWS, D), jnp.float32),
          pltpu.SemaphoreType.DMA,
          pltpu.SemaphoreType.DMA,
          pltpu.SemaphoreType.DMA,
          pltpu.SemaphoreType.DMA,
          pltpu.SemaphoreType.DMA,
          pltpu.SemaphoreType.DMA,
      ],
  )(wfix, wvar, fix_gidx, var_gidx)
  return out.reshape(B, N_OUT * D)


# speedup vs baseline: 1.0006x; 1.0006x over previous
import jax
import jax.numpy as jnp
from jax import lax
from jax.experimental import pallas as pl
from jax.experimental.pallas import tpu as pltpu
from jax.experimental.pallas import tpu_sc as plsc

B = 16384
N_FIX = 26
N_VAR = 4
VOCAB = 100000
L = 50
D = 32
N_OUT = N_FIX + N_VAR

NC = 2
NS = 16
NW = NC * NS

R = 8
BLKS = B // R
BPW = BLKS // NW
PAIRS = R * N_VAR
VROWS = R * N_VAR * L
VG = 13
VGN = 128
VPAD = VG * VGN
OROWS = R * N_OUT
FG = 2
FGN = OROWS // FG


def _sc_body(wfix_hbm, wvar_hbm, fixidx_hbm, varidx_hbm, out_hbm,
             fixidx_v, varidx_v, var_buf, outbuf,
             isem0, isem1, gsem0, gsem1, osem0, osem1):
  wid = lax.axis_index("s") * NC + lax.axis_index("c")
  blk0 = wid * BPW
  isems = (isem0, isem1)
  gsems = (gsem0, gsem1)
  osems = (osem0, osem1)
  inv_l = jnp.float32(1.0 / L)

  def stage_idx(blk, s):
    pltpu.async_copy(fixidx_hbm.at[pl.ds(blk * OROWS, OROWS)],
                     fixidx_v.at[s], isems[s])
    pltpu.async_copy(varidx_hbm.at[blk], varidx_v.at[s], isems[s])

  def wait_idx(s):
    pltpu.make_async_copy(fixidx_hbm.at[pl.ds(0, OROWS)], fixidx_v.at[s],
                          isems[s]).wait()
    pltpu.make_async_copy(varidx_hbm.at[0], varidx_v.at[s], isems[s]).wait()

  def fire_gathers(s):
    for h in range(FG):
      pltpu.async_copy(
          wfix_hbm.at[fixidx_v.at[s, pl.ds(h * FGN, FGN)]],
          outbuf.at[s, pl.ds(h * FGN, FGN)], gsems[s])
    for g in range(VG):
      pltpu.async_copy(
          wvar_hbm.at[varidx_v.at[s, g]],
          var_buf.at[s, pl.ds(g * VGN, VGN)], gsems[s])

  def drain_gathers(s):
    for h in range(FG):
      pltpu.make_async_copy(
          wfix_hbm.at[fixidx_v.at[s, pl.ds(0, FGN)]],
          outbuf.at[s, pl.ds(0, FGN)], gsems[s]).wait()
    for g in range(VG):
      pltpu.make_async_copy(
          wvar_hbm.at[varidx_v.at[s, 0]],
          var_buf.at[s, pl.ds(0, VGN)], gsems[s]).wait()

  def pool(s):
    for p0 in range(0, PAIRS, 8):
      def red(r, acc):
        new = []
        for k in range(8):
          p = p0 + k
          new.append(acc[2 * k] + var_buf[s, p * L + r, pl.ds(0, 16)])
          new.append(acc[2 * k + 1] + var_buf[s, p * L + r, pl.ds(16, 16)])
        return tuple(new)

      acc = lax.fori_loop(
          0, L, red, tuple(jnp.zeros((16,), jnp.float32) for _ in range(16)))
      for k in range(8):
        p = p0 + k
        row = (p // N_VAR) * N_OUT + N_FIX + (p % N_VAR)
        outbuf[s, row, pl.ds(0, 16)] = acc[2 * k] * inv_l
        outbuf[s, row, pl.ds(16, 16)] = acc[2 * k + 1] * inv_l

  def issue_out(blk, s):
    pltpu.async_copy(outbuf.at[s], out_hbm.at[pl.ds(blk * OROWS, OROWS)],
                     osems[s])

  def wait_out(s):
    pltpu.make_async_copy(outbuf.at[s], out_hbm.at[pl.ds(0, OROWS)],
                          osems[s]).wait()

  def body(i, carry):
    blk = blk0 + i
    stage_idx(blk, 0)
    wait_idx(0)
    fire_gathers(0)
    drain_gathers(0)
    pool(0)
    issue_out(blk, 0)
    wait_out(0)
    return carry

  lax.fori_loop(0, BPW, body, 0, unroll=False)


@jax.jit
def kernel(x_fix, x_varlen, W_fix, W_var):
  wfix = W_fix.reshape(N_FIX * VOCAB, D)
  wvar = W_var.reshape(N_VAR * VOCAB, D)
  fix_gidx = (x_fix.astype(jnp.int32)
              + (jnp.arange(N_FIX, dtype=jnp.int32) * VOCAB)[None, :])
  fix_gidx = jnp.pad(fix_gidx.reshape(BLKS, R, N_FIX),
                     ((0, 0), (0, 0), (0, N_OUT - N_FIX)))
  fix_gidx = fix_gidx.reshape(B * N_OUT)
  var_gidx = (x_varlen.astype(jnp.int32)
              + (jnp.arange(N_VAR, dtype=jnp.int32) * VOCAB)[None, :, None])
  var_gidx = var_gidx.reshape(BLKS, VROWS)
  var_gidx = jnp.pad(var_gidx, ((0, 0), (0, VPAD - VROWS)))
  var_gidx = var_gidx.reshape(BLKS, VG, VGN)

  mesh = plsc.VectorSubcoreMesh(core_axis_name="c", subcore_axis_name="s")
  out = pl.kernel(
      _sc_body,
      out_type=jax.ShapeDtypeStruct((B * N_OUT, D), jnp.float32),
      mesh=mesh,
      compiler_params=pltpu.CompilerParams(use_tc_tiling_on_sc=False),
      scratch_types=[
          pltpu.VMEM((2, OROWS), jnp.int32),
          pltpu.VMEM((2, VG, VGN), jnp.int32),
          pltpu.VMEM((2, VPAD, D), jnp.float32),
          pltpu.VMEM((2, OROWS, D), jnp.float32),
          pltpu.SemaphoreType.DMA,
          pltpu.SemaphoreType.DMA,
          pltpu.SemaphoreType.DMA,
          pltpu.SemaphoreType.DMA,
          pltpu.SemaphoreType.DMA,
          pltpu.SemaphoreType.DMA,
      ],
  )(wfix, wvar, fix_gidx, var_gidx)
  return out.reshape(B, N_OUT * D)


# speedup vs baseline: 1.0013x; 1.0007x over previous
import jax
import jax.numpy as jnp
from jax import lax
from jax.experimental import pallas as pl
from jax.experimental.pallas import tpu as pltpu
from jax.experimental.pallas import tpu_sc as plsc

B = 16384
N_FIX = 26
N_VAR = 4
VOCAB = 100000
L = 50
D = 32
N_OUT = N_FIX + N_VAR

NC = 2
NS = 16
NW = NC * NS

R = 8
BLKS = B // R
BPW = BLKS // NW
PAIRS = R * N_VAR
VROWS = R * N_VAR * L
VG = 13
VGN = 128
VPAD = VG * VGN
OROWS = R * N_OUT
FG = 2
FGN = OROWS // FG


def _sc_body(wfix_hbm, wvar_hbm, fixidx_hbm, varidx_hbm, out_hbm,
             fixidx_v, varidx_v, var_buf, outbuf,
             isem0, isem1, gsem0, gsem1, osem0, osem1):
  wid = lax.axis_index("s") * NC + lax.axis_index("c")
  blk0 = wid * BPW
  isems = (isem0, isem1)
  gsems = (gsem0, gsem1)
  osems = (osem0, osem1)
  inv_l = jnp.float32(1.0 / L)

  def stage_idx(blk, s):
    pltpu.async_copy(fixidx_hbm.at[pl.ds(blk * OROWS, OROWS)],
                     fixidx_v.at[s], isems[s])
    pltpu.async_copy(varidx_hbm.at[blk], varidx_v.at[s], isems[s])

  def wait_idx(s):
    pltpu.make_async_copy(fixidx_hbm.at[pl.ds(0, OROWS)], fixidx_v.at[s],
                          isems[s]).wait()
    pltpu.make_async_copy(varidx_hbm.at[0], varidx_v.at[s], isems[s]).wait()

  def fire_gathers(s):
    for h in range(FG):
      pltpu.async_copy(
          wfix_hbm.at[fixidx_v.at[s, pl.ds(h * FGN, FGN)]],
          outbuf.at[s, pl.ds(h * FGN, FGN)], gsems[s])
    for g in range(VG):
      pltpu.async_copy(
          wvar_hbm.at[varidx_v.at[s, g]],
          var_buf.at[s, pl.ds(g * VGN, VGN)], gsems[s])

  def drain_gathers(s):
    for h in range(FG):
      pltpu.make_async_copy(
          wfix_hbm.at[fixidx_v.at[s, pl.ds(0, FGN)]],
          outbuf.at[s, pl.ds(0, FGN)], gsems[s]).wait()
    for g in range(VG):
      pltpu.make_async_copy(
          wvar_hbm.at[varidx_v.at[s, 0]],
          var_buf.at[s, pl.ds(0, VGN)], gsems[s]).wait()

  def pool(s):
    for p0 in range(0, PAIRS, 8):
      def red(r, acc):
        new = []
        for k in range(8):
          p = p0 + k
          new.append(acc[2 * k] + var_buf[s, p * L + r, pl.ds(0, 16)])
          new.append(acc[2 * k + 1] + var_buf[s, p * L + r, pl.ds(16, 16)])
        return tuple(new)

      acc = lax.fori_loop(
          0, L, red, tuple(jnp.zeros((16,), jnp.float32) for _ in range(16)))
      for k in range(8):
        p = p0 + k
        row = (p // N_VAR) * N_OUT + N_FIX + (p % N_VAR)
        outbuf[s, row, pl.ds(0, 16)] = acc[2 * k] * inv_l
        outbuf[s, row, pl.ds(16, 16)] = acc[2 * k + 1] * inv_l

  def issue_out(blk, s):
    pltpu.async_copy(outbuf.at[s], out_hbm.at[pl.ds(blk * OROWS, OROWS)],
                     osems[s])

  def wait_out(s):
    pltpu.make_async_copy(outbuf.at[s], out_hbm.at[pl.ds(0, OROWS)],
                          osems[s]).wait()

  def body(i, carry):
    blk = blk0 + i
    c1 = pltpu.async_copy(fixidx_hbm.at[pl.ds(blk * OROWS, OROWS)],
                          fixidx_v.at[0], isems[0])
    c2 = pltpu.async_copy(varidx_hbm.at[blk], varidx_v.at[0], isems[0])
    c1.wait()
    c2.wait()
    copies = []
    for h in range(FG):
      copies.append(pltpu.async_copy(
          wfix_hbm.at[fixidx_v.at[0, pl.ds(h * FGN, FGN)]],
          outbuf.at[0, pl.ds(h * FGN, FGN)], gsems[0]))
    for g in range(VG):
      copies.append(pltpu.async_copy(
          wvar_hbm.at[varidx_v.at[0, g]],
          var_buf.at[0, pl.ds(g * VGN, VGN)], gsems[0]))
    for c in copies:
      c.wait()
    pool(0)
    co = pltpu.async_copy(outbuf.at[0],
                          out_hbm.at[pl.ds(blk * OROWS, OROWS)], osems[0])
    co.wait()
    return carry

  lax.fori_loop(0, BPW, body, 0, unroll=False)


@jax.jit
def kernel(x_fix, x_varlen, W_fix, W_var):
  wfix = W_fix.reshape(N_FIX * VOCAB, D)
  wvar = W_var.reshape(N_VAR * VOCAB, D)
  fix_gidx = (x_fix.astype(jnp.int32)
              + (jnp.arange(N_FIX, dtype=jnp.int32) * VOCAB)[None, :])
  fix_gidx = jnp.pad(fix_gidx.reshape(BLKS, R, N_FIX),
                     ((0, 0), (0, 0), (0, N_OUT - N_FIX)))
  fix_gidx = fix_gidx.reshape(B * N_OUT)
  var_gidx = (x_varlen.astype(jnp.int32)
              + (jnp.arange(N_VAR, dtype=jnp.int32) * VOCAB)[None, :, None])
  var_gidx = var_gidx.reshape(BLKS, VROWS)
  var_gidx = jnp.pad(var_gidx, ((0, 0), (0, VPAD - VROWS)))
  var_gidx = var_gidx.reshape(BLKS, VG, VGN)

  mesh = plsc.VectorSubcoreMesh(core_axis_name="c", subcore_axis_name="s")
  out = pl.kernel(
      _sc_body,
      out_type=jax.ShapeDtypeStruct((B * N_OUT, D), jnp.float32),
      mesh=mesh,
      compiler_params=pltpu.CompilerParams(use_tc_tiling_on_sc=False),
      scratch_types=[
          pltpu.VMEM((2, OROWS), jnp.int32),
          pltpu.VMEM((2, VG, VGN), jnp.int32),
          pltpu.VMEM((2, VPAD, D), jnp.float32),
          pltpu.VMEM((2, OROWS, D), jnp.float32),
          pltpu.SemaphoreType.DMA,
          pltpu.SemaphoreType.DMA,
          pltpu.SemaphoreType.DMA,
          pltpu.SemaphoreType.DMA,
          pltpu.SemaphoreType.DMA,
          pltpu.SemaphoreType.DMA,
      ],
  )(wfix, wvar, fix_gidx, var_gidx)
  return out.reshape(B, N_OUT * D)


# speedup vs baseline: 1.6376x; 1.6355x over previous
import jax
import jax.numpy as jnp
from jax import lax
from jax.experimental import pallas as pl
from jax.experimental.pallas import tpu as pltpu
from jax.experimental.pallas import tpu_sc as plsc

B = 16384
N_FIX = 26
N_VAR = 4
VOCAB = 100000
L = 50
D = 32
N_OUT = N_FIX + N_VAR

NC = 2
NS = 16
NW = NC * NS

R = 8
BLKS = B // R
BPW = BLKS // NW
PAIRS = R * N_VAR
VROWS = R * N_VAR * L
VG = 13
VGN = 128
VPAD = VG * VGN
OROWS = R * N_OUT
FG = 2
FGN = OROWS // FG


def _sc_body(wfix_hbm, wvar_hbm, fixidx_hbm, varidx_hbm, out_hbm,
             fixidx_v, varidx_v, var_buf, outbuf,
             isem0, isem1, gsem0, gsem1, osem0, osem1):
  wid = lax.axis_index("s") * NC + lax.axis_index("c")
  blk0 = wid * BPW
  isems = (isem0, isem1)
  gsems = (gsem0, gsem1)
  osems = (osem0, osem1)
  inv_l = jnp.float32(1.0 / L)

  def stage_idx(blk, s):
    pltpu.async_copy(fixidx_hbm.at[pl.ds(blk * OROWS, OROWS)],
                     fixidx_v.at[s], isems[s])
    pltpu.async_copy(varidx_hbm.at[blk], varidx_v.at[s], isems[s])

  def wait_idx(s):
    pltpu.make_async_copy(fixidx_hbm.at[pl.ds(0, OROWS)], fixidx_v.at[s],
                          isems[s]).wait()
    pltpu.make_async_copy(varidx_hbm.at[0], varidx_v.at[s], isems[s]).wait()

  def fire_gathers(s):
    for h in range(FG):
      pltpu.async_copy(
          wfix_hbm.at[fixidx_v.at[s, pl.ds(h * FGN, FGN)]],
          outbuf.at[s, pl.ds(h * FGN, FGN)], gsems[s])
    for g in range(VG):
      pltpu.async_copy(
          wvar_hbm.at[varidx_v.at[s, g]],
          var_buf.at[s, pl.ds(g * VGN, VGN)], gsems[s])

  def drain_gathers(s):
    for h in range(FG):
      pltpu.make_async_copy(
          wfix_hbm.at[fixidx_v.at[s, pl.ds(0, FGN)]],
          outbuf.at[s, pl.ds(0, FGN)], gsems[s]).wait()
    for g in range(VG):
      pltpu.make_async_copy(
          wvar_hbm.at[varidx_v.at[s, 0]],
          var_buf.at[s, pl.ds(0, VGN)], gsems[s]).wait()

  def pool(s):
    for p0 in range(0, PAIRS, 8):
      def red(r, acc):
        new = []
        for k in range(8):
          p = p0 + k
          new.append(acc[2 * k] + var_buf[s, p * L + r, pl.ds(0, 16)])
          new.append(acc[2 * k + 1] + var_buf[s, p * L + r, pl.ds(16, 16)])
        return tuple(new)

      acc = lax.fori_loop(
          0, L, red, tuple(jnp.zeros((16,), jnp.float32) for _ in range(16)))
      for k in range(8):
        p = p0 + k
        row = (p // N_VAR) * N_OUT + N_FIX + (p % N_VAR)
        outbuf[s, row, pl.ds(0, 16)] = acc[2 * k] * inv_l
        outbuf[s, row, pl.ds(16, 16)] = acc[2 * k + 1] * inv_l

  def issue_out(blk, s):
    pltpu.async_copy(outbuf.at[s], out_hbm.at[pl.ds(blk * OROWS, OROWS)],
                     osems[s])

  def wait_out(s):
    pltpu.make_async_copy(outbuf.at[s], out_hbm.at[pl.ds(0, OROWS)],
                          osems[s]).wait()

  def body(i, carry):
    blk = blk0 + i
    c1 = pltpu.async_copy(fixidx_hbm.at[pl.ds(blk * OROWS, OROWS)],
                          fixidx_v.at[0], isems[0])
    c2 = pltpu.async_copy(varidx_hbm.at[blk], varidx_v.at[0], isems[0])
    c1.wait()
    c2.wait()
    copies = []
    for h in range(FG):
      copies.append(pltpu.async_copy(
          wfix_hbm.at[fixidx_v.at[0, pl.ds(h * FGN, FGN)]],
          outbuf.at[0, pl.ds(h * FGN, FGN)], gsems[0]))
    for g in range(VG):
      copies.append(pltpu.async_copy(
          wvar_hbm.at[varidx_v.at[0, g]],
          var_buf.at[0, pl.ds(g * VGN, VGN)], gsems[0]))
    for c in copies:
      c.wait()
    pool(0)
    co = pltpu.async_copy(outbuf.at[0],
                          out_hbm.at[pl.ds(blk * OROWS, OROWS)], osems[0])
    co.wait()
    return carry

  lax.fori_loop(0, BPW, body, 0, unroll=False)


@jax.jit
def kernel(x_fix, x_varlen, W_fix, W_var):
  wfix = W_fix.reshape(N_FIX * VOCAB, D)
  wvar = W_var.reshape(N_VAR * VOCAB, D)
  fix_gidx = (x_fix.astype(jnp.int32)
              + (jnp.arange(N_FIX, dtype=jnp.int32) * VOCAB)[None, :])
  fix_gidx = jnp.pad(fix_gidx.reshape(BLKS, R, N_FIX),
                     ((0, 0), (0, 0), (0, N_OUT - N_FIX)), mode="wrap")
  fix_gidx = fix_gidx.reshape(B * N_OUT)
  var_gidx = (x_varlen.astype(jnp.int32)
              + (jnp.arange(N_VAR, dtype=jnp.int32) * VOCAB)[None, :, None])
  var_gidx = var_gidx.reshape(BLKS, VROWS)
  var_gidx = jnp.pad(var_gidx, ((0, 0), (0, VPAD - VROWS)), mode="wrap")
  var_gidx = var_gidx.reshape(BLKS, VG, VGN)

  mesh = plsc.VectorSubcoreMesh(core_axis_name="c", subcore_axis_name="s")
  out = pl.kernel(
      _sc_body,
      out_type=jax.ShapeDtypeStruct((B * N_OUT, D), jnp.float32),
      mesh=mesh,
      compiler_params=pltpu.CompilerParams(use_tc_tiling_on_sc=False),
      scratch_types=[
          pltpu.VMEM((2, OROWS), jnp.int32),
          pltpu.VMEM((2, VG, VGN), jnp.int32),
          pltpu.VMEM((2, VPAD, D), jnp.float32),
          pltpu.VMEM((2, OROWS, D), jnp.float32),
          pltpu.SemaphoreType.DMA,
          pltpu.SemaphoreType.DMA,
          pltpu.SemaphoreType.DMA,
          pltpu.SemaphoreType.DMA,
          pltpu.SemaphoreType.DMA,
          pltpu.SemaphoreType.DMA,
      ],
  )(wfix, wvar, fix_gidx, var_gidx)
  return out.reshape(B, N_OUT * D)


# speedup vs baseline: 1.8268x; 1.1155x over previous
import jax
import jax.numpy as jnp
from jax import lax
from jax.experimental import pallas as pl
from jax.experimental.pallas import tpu as pltpu
from jax.experimental.pallas import tpu_sc as plsc

B = 16384
N_FIX = 26
N_VAR = 4
VOCAB = 100000
L = 50
D = 32
N_OUT = N_FIX + N_VAR

NC = 2
NS = 16
NW = NC * NS

R = 8
BLKS = B // R
BPW = BLKS // NW
PAIRS = R * N_VAR
VROWS = R * N_VAR * L
VG = 13
VGN = 128
VPAD = VG * VGN
OROWS = R * N_OUT
FG = 2
FGN = OROWS // FG


def _sc_body(wfix_hbm, wvar_hbm, fixidx_hbm, varidx_hbm, out_hbm,
             fixidx_v, varidx_v, var_buf, outbuf,
             isem0, isem1, gsem0, gsem1, osem0, osem1):
  wid = lax.axis_index("s") * NC + lax.axis_index("c")
  blk0 = wid * BPW
  isems = (isem0, isem1)
  gsems = (gsem0, gsem1)
  osems = (osem0, osem1)
  inv_l = jnp.float32(1.0 / L)

  def stage_idx(blk, s):
    pltpu.async_copy(fixidx_hbm.at[pl.ds(blk * OROWS, OROWS)],
                     fixidx_v.at[s], isems[s])
    pltpu.async_copy(varidx_hbm.at[blk], varidx_v.at[s], isems[s])

  def wait_idx(s):
    pltpu.make_async_copy(fixidx_hbm.at[pl.ds(0, OROWS)], fixidx_v.at[s],
                          isems[s]).wait()
    pltpu.make_async_copy(varidx_hbm.at[0], varidx_v.at[s], isems[s]).wait()

  def fire_gathers(s):
    for h in range(FG):
      pltpu.async_copy(
          wfix_hbm.at[fixidx_v.at[s, pl.ds(h * FGN, FGN)]],
          outbuf.at[s, pl.ds(h * FGN, FGN)], gsems[s])
    for g in range(VG):
      pltpu.async_copy(
          wvar_hbm.at[varidx_v.at[s, g]],
          var_buf.at[s, pl.ds(g * VGN, VGN)], gsems[s])

  def drain_gathers(s):
    for h in range(FG):
      pltpu.make_async_copy(
          wfix_hbm.at[fixidx_v.at[s, pl.ds(0, FGN)]],
          outbuf.at[s, pl.ds(0, FGN)], gsems[s]).wait()
    for g in range(VG):
      pltpu.make_async_copy(
          wvar_hbm.at[varidx_v.at[s, 0]],
          var_buf.at[s, pl.ds(0, VGN)], gsems[s]).wait()

  def pool(s):
    for p0 in range(0, PAIRS, 8):
      def red(r, acc):
        new = []
        for k in range(8):
          p = p0 + k
          new.append(acc[2 * k] + var_buf[s, p * L + r, pl.ds(0, 16)])
          new.append(acc[2 * k + 1] + var_buf[s, p * L + r, pl.ds(16, 16)])
        return tuple(new)

      acc = lax.fori_loop(
          0, L, red, tuple(jnp.zeros((16,), jnp.float32) for _ in range(16)))
      for k in range(8):
        p = p0 + k
        row = (p // N_VAR) * N_OUT + N_FIX + (p % N_VAR)
        outbuf[s, row, pl.ds(0, 16)] = acc[2 * k] * inv_l
        outbuf[s, row, pl.ds(16, 16)] = acc[2 * k + 1] * inv_l

  def issue_out(blk, s):
    pltpu.async_copy(outbuf.at[s], out_hbm.at[pl.ds(blk * OROWS, OROWS)],
                     osems[s])

  def wait_out(s):
    pltpu.make_async_copy(outbuf.at[s], out_hbm.at[pl.ds(0, OROWS)],
                          osems[s]).wait()

  stage_idx(blk0, 0)

  def body(i2, carry):
    e = blk0 + 2 * i2
    o = e + 1

    wait_idx(0)

    @pl.when(i2 >= 1)
    def _():
      wait_out(0)

    fire_gathers(0)

    @pl.when(i2 >= 1)
    def _():
      drain_gathers(1)
      stage_idx(o, 1)
      pool(1)
      issue_out(o - 2, 1)

    @pl.when(i2 == 0)
    def _():
      stage_idx(o, 1)

    wait_idx(1)

    @pl.when(i2 >= 1)
    def _():
      wait_out(1)

    fire_gathers(1)
    drain_gathers(0)

    @pl.when(i2 < BPW // 2 - 1)
    def _():
      stage_idx(e + 2, 0)

    pool(0)
    issue_out(e, 0)
    return carry

  lax.fori_loop(0, BPW // 2, body, 0, unroll=False)

  drain_gathers(1)
  pool(1)
  issue_out(blk0 + BPW - 1, 1)
  wait_out(0)
  wait_out(1)


@jax.jit
def kernel(x_fix, x_varlen, W_fix, W_var):
  wfix = W_fix.reshape(N_FIX * VOCAB, D)
  wvar = W_var.reshape(N_VAR * VOCAB, D)
  fix_gidx = (x_fix.astype(jnp.int32)
              + (jnp.arange(N_FIX, dtype=jnp.int32) * VOCAB)[None, :])
  fix_gidx = jnp.pad(fix_gidx.reshape(BLKS, R, N_FIX),
                     ((0, 0), (0, 0), (0, N_OUT - N_FIX)), mode="wrap")
  fix_gidx = fix_gidx.reshape(B * N_OUT)
  var_gidx = (x_varlen.astype(jnp.int32)
              + (jnp.arange(N_VAR, dtype=jnp.int32) * VOCAB)[None, :, None])
  var_gidx = var_gidx.reshape(BLKS, VROWS)
  var_gidx = jnp.pad(var_gidx, ((0, 0), (0, VPAD - VROWS)), mode="wrap")
  var_gidx = var_gidx.reshape(BLKS, VG, VGN)

  mesh = plsc.VectorSubcoreMesh(core_axis_name="c", subcore_axis_name="s")
  out = pl.kernel(
      _sc_body,
      out_type=jax.ShapeDtypeStruct((B * N_OUT, D), jnp.float32),
      mesh=mesh,
      compiler_params=pltpu.CompilerParams(use_tc_tiling_on_sc=False),
      scratch_types=[
          pltpu.VMEM((2, OROWS), jnp.int32),
          pltpu.VMEM((2, VG, VGN), jnp.int32),
          pltpu.VMEM((2, VPAD, D), jnp.float32),
          pltpu.VMEM((2, OROWS, D), jnp.float32),
          pltpu.SemaphoreType.DMA,
          pltpu.SemaphoreType.DMA,
          pltpu.SemaphoreType.DMA,
          pltpu.SemaphoreType.DMA,
          pltpu.SemaphoreType.DMA,
          pltpu.SemaphoreType.DMA,
      ],
  )(wfix, wvar, fix_gidx, var_gidx)
  return out.reshape(B, N_OUT * D)


# speedup vs baseline: 1.8293x; 1.0014x over previous
import jax
import jax.numpy as jnp
from jax import lax
from jax.experimental import pallas as pl
from jax.experimental.pallas import tpu as pltpu
from jax.experimental.pallas import tpu_sc as plsc

B = 16384
N_FIX = 26
N_VAR = 4
VOCAB = 100000
L = 50
D = 32
N_OUT = N_FIX + N_VAR

NC = 2
NS = 16
NW = NC * NS

R = 8
BLKS = B // R
BPW = BLKS // NW
PAIRS = R * N_VAR
VROWS = R * N_VAR * L
VG = 13
VGN = 128
VPAD = VG * VGN
OROWS = R * N_OUT
FG = 2
FGN = OROWS // FG


def _sc_body(wfix3_hbm, wvar3_hbm, fixidx_hbm, varidx_hbm, out_hbm,
             fixidx_v, varidx_v, var_buf, outbuf,
             isem0, isem1, gsem0, gsem1, osem0, osem1):
  wfix_hbm = wfix3_hbm.at[0]
  wvar_hbm = wvar3_hbm.at[0]
  wid = lax.axis_index("s") * NC + lax.axis_index("c")
  blk0 = wid * BPW
  isems = (isem0, isem1)
  gsems = (gsem0, gsem1)
  osems = (osem0, osem1)
  inv_l = jnp.float32(1.0 / L)

  def stage_idx(blk, s):
    pltpu.async_copy(fixidx_hbm.at[pl.ds(blk * OROWS, OROWS)],
                     fixidx_v.at[s], isems[s])
    pltpu.async_copy(varidx_hbm.at[blk], varidx_v.at[s], isems[s])

  def wait_idx(s):
    pltpu.make_async_copy(fixidx_hbm.at[pl.ds(0, OROWS)], fixidx_v.at[s],
                          isems[s]).wait()
    pltpu.make_async_copy(varidx_hbm.at[0], varidx_v.at[s], isems[s]).wait()

  def fire_gathers(s):
    for h in range(FG):
      pltpu.async_copy(
          wfix_hbm.at[fixidx_v.at[s, pl.ds(h * FGN, FGN)]],
          outbuf.at[s, pl.ds(h * FGN, FGN)], gsems[s])
    for g in range(VG):
      pltpu.async_copy(
          wvar_hbm.at[varidx_v.at[s, g]],
          var_buf.at[s, pl.ds(g * VGN, VGN)], gsems[s])

  def drain_gathers(s):
    for h in range(FG):
      pltpu.make_async_copy(
          wfix_hbm.at[fixidx_v.at[s, pl.ds(0, FGN)]],
          outbuf.at[s, pl.ds(0, FGN)], gsems[s]).wait()
    for g in range(VG):
      pltpu.make_async_copy(
          wvar_hbm.at[varidx_v.at[s, 0]],
          var_buf.at[s, pl.ds(0, VGN)], gsems[s]).wait()

  def pool(s):
    for p0 in range(0, PAIRS, 8):
      def red(r, acc):
        new = []
        for k in range(8):
          p = p0 + k
          new.append(acc[2 * k] + var_buf[s, p * L + r, pl.ds(0, 16)])
          new.append(acc[2 * k + 1] + var_buf[s, p * L + r, pl.ds(16, 16)])
        return tuple(new)

      acc = lax.fori_loop(
          0, L, red, tuple(jnp.zeros((16,), jnp.float32) for _ in range(16)))
      for k in range(8):
        p = p0 + k
        row = (p // N_VAR) * N_OUT + N_FIX + (p % N_VAR)
        outbuf[s, row, pl.ds(0, 16)] = acc[2 * k] * inv_l
        outbuf[s, row, pl.ds(16, 16)] = acc[2 * k + 1] * inv_l

  def issue_out(blk, s):
    pltpu.async_copy(outbuf.at[s], out_hbm.at[pl.ds(blk * OROWS, OROWS)],
                     osems[s])

  def wait_out(s):
    pltpu.make_async_copy(outbuf.at[s], out_hbm.at[pl.ds(0, OROWS)],
                          osems[s]).wait()

  stage_idx(blk0, 0)

  def body(i2, carry):
    e = blk0 + 2 * i2
    o = e + 1

    wait_idx(0)

    @pl.when(i2 >= 1)
    def _():
      wait_out(0)

    fire_gathers(0)

    @pl.when(i2 >= 1)
    def _():
      drain_gathers(1)
      stage_idx(o, 1)
      pool(1)
      issue_out(o - 2, 1)

    @pl.when(i2 == 0)
    def _():
      stage_idx(o, 1)

    wait_idx(1)

    @pl.when(i2 >= 1)
    def _():
      wait_out(1)

    fire_gathers(1)
    drain_gathers(0)

    @pl.when(i2 < BPW // 2 - 1)
    def _():
      stage_idx(e + 2, 0)

    pool(0)
    issue_out(e, 0)
    return carry

  lax.fori_loop(0, BPW // 2, body, 0, unroll=False)

  drain_gathers(1)
  pool(1)
  issue_out(blk0 + BPW - 1, 1)
  wait_out(0)
  wait_out(1)


@jax.jit
def kernel(x_fix, x_varlen, W_fix, W_var):
  fix_gidx = (x_fix.astype(jnp.int32)
              + (jnp.arange(N_FIX, dtype=jnp.int32) * VOCAB)[None, :])
  fix_gidx = jnp.pad(fix_gidx.reshape(BLKS, R, N_FIX),
                     ((0, 0), (0, 0), (0, N_OUT - N_FIX)), mode="wrap")
  fix_gidx = fix_gidx.reshape(B * N_OUT)
  var_gidx = (x_varlen.astype(jnp.int32)
              + (jnp.arange(N_VAR, dtype=jnp.int32) * VOCAB)[None, :, None])
  var_gidx = var_gidx.reshape(BLKS, VROWS)
  var_gidx = jnp.pad(var_gidx, ((0, 0), (0, VPAD - VROWS)), mode="wrap")
  var_gidx = var_gidx.reshape(BLKS, VG, VGN)

  mesh = plsc.VectorSubcoreMesh(core_axis_name="c", subcore_axis_name="s")
  out = pl.kernel(
      _sc_body,
      out_type=jax.ShapeDtypeStruct((B * N_OUT, D), jnp.float32),
      mesh=mesh,
      compiler_params=pltpu.CompilerParams(use_tc_tiling_on_sc=False,
                                           disable_bounds_checks=True),
      scratch_types=[
          pltpu.VMEM((2, OROWS), jnp.int32),
          pltpu.VMEM((2, VG, VGN), jnp.int32),
          pltpu.VMEM((2, VPAD, D), jnp.float32),
          pltpu.VMEM((2, OROWS, D), jnp.float32),
          pltpu.SemaphoreType.DMA,
          pltpu.SemaphoreType.DMA,
          pltpu.SemaphoreType.DMA,
          pltpu.SemaphoreType.DMA,
          pltpu.SemaphoreType.DMA,
          pltpu.SemaphoreType.DMA,
      ],
  )(W_fix, W_var, fix_gidx, var_gidx)
  return out.reshape(B, N_OUT * D)
